# baseline (speedup 1.0000x reference)
import jax
import jax.numpy as jnp
from jax.experimental import pallas as pl


def _copy_kernel(x_ref, a_ref):
    a_ref[...] = x_ref[...]


def kernel(x, mask):
    B, S, D = x.shape
    blk = (1, S, D)
    spec = pl.BlockSpec(blk, lambda i: (i, 0, 0))
    out = pl.pallas_call(
        _copy_kernel,
        grid=(B,),
        in_specs=[spec],
        out_specs=spec,
        out_shape=jax.ShapeDtypeStruct(x.shape, x.dtype),
    )(x)
    return (out, out)


# speedup vs baseline: 1.3642x; 1.3642x over previous
import jax
import jax.numpy as jnp
from jax.experimental import pallas as pl
from jax.experimental.pallas import tpu as pltpu


def _copy2_kernel(x_ref, a_ref, b_ref):
    v = x_ref[...]
    a_ref[...] = v
    b_ref[...] = v


def kernel(x, mask):
    B, S, D = x.shape
    blk = (1, S, D)
    spec = pl.BlockSpec(blk, lambda i: (i, 0, 0))
    out = pl.pallas_call(
        _copy2_kernel,
        grid=(B,),
        in_specs=[spec],
        out_specs=[spec, spec],
        out_shape=[
            jax.ShapeDtypeStruct(x.shape, x.dtype),
            jax.ShapeDtypeStruct(x.shape, x.dtype),
        ],
        compiler_params=pltpu.CompilerParams(
            dimension_semantics=("parallel",),
        ),
    )(x)
    return (out[0], out[1])
